# baseline (device time: 15452 ns/iter reference)
import jax
import jax.numpy as jnp
from jax import lax
from jax.experimental import pallas as pl
from jax.experimental.pallas import tpu as pltpu

N_DEV = 4
E_PER = 2
N_EXP = N_DEV * E_PER


def kernel(x, router_W, route_idx, expert_W):
    n_tok, d = x.shape
    e_per, _, h = expert_W.shape

    def body(x_ref, rw_ref, idx_ref, ew_ref, out_ref,
             comm_ref, send_sems, recv_sems):
        my = lax.axis_index("i")
        left = lax.rem(my + N_DEV - 1, N_DEV)
        right = lax.rem(my + 1, N_DEV)

        barrier_sem = pltpu.get_barrier_semaphore()
        for nbr in (left, right):
            pl.semaphore_signal(
                barrier_sem, inc=1,
                device_id=(nbr,), device_id_type=pl.DeviceIdType.MESH,
            )
        pl.semaphore_wait(barrier_sem, 2)

        comm_ref[0, :, :, :] = ew_ref[...].astype(jnp.bfloat16)

        scores = jnp.dot(x_ref[...], rw_ref[...],
                         preferred_element_type=jnp.float32)
        probs = jax.nn.softmax(scores, axis=-1)
        eids = lax.broadcasted_iota(jnp.int32, (n_tok, N_EXP), 1)
        mask = (eids == idx_ref[:, 0:1]) | (eids == idx_ref[:, 1:2])
        mp = jnp.where(mask, probs, 0.0)
        w = mp / jnp.sum(mp, axis=-1, keepdims=True)

        xb = x_ref[...].astype(jnp.bfloat16)

        def contrib(origin, shard):
            g0 = jnp.sum(jnp.where(eids == 2 * origin, w, 0.0),
                         axis=-1, keepdims=True)
            g1 = jnp.sum(jnp.where(eids == 2 * origin + 1, w, 0.0),
                         axis=-1, keepdims=True)
            y0 = jnp.dot(xb, shard[0], preferred_element_type=jnp.float32)
            y1 = jnp.dot(xb, shard[1], preferred_element_type=jnp.float32)
            return g0 * y0 + g1 * y1

        acc = jnp.zeros((n_tok, h), jnp.float32)
        for hop in range(N_DEV - 1):
            rdma = pltpu.make_async_remote_copy(
                src_ref=comm_ref.at[hop],
                dst_ref=comm_ref.at[hop + 1],
                send_sem=send_sems.at[hop],
                recv_sem=recv_sems.at[hop],
                device_id=(right,),
                device_id_type=pl.DeviceIdType.MESH,
            )
            rdma.start()
            acc = acc + contrib(lax.rem(my - hop + N_DEV, N_DEV),
                                comm_ref[hop])
            rdma.wait()
        acc = acc + contrib(lax.rem(my + 1, N_DEV), comm_ref[N_DEV - 1])
        out_ref[...] = acc

    return pl.pallas_call(
        body,
        out_shape=jax.ShapeDtypeStruct((n_tok, h), jnp.float32),
        in_specs=[pl.BlockSpec(memory_space=pltpu.VMEM)] * 4,
        out_specs=pl.BlockSpec(memory_space=pltpu.VMEM),
        scratch_shapes=[
            pltpu.VMEM((N_DEV, e_per, d, h), jnp.bfloat16),
            pltpu.SemaphoreType.DMA((N_DEV - 1,)),
            pltpu.SemaphoreType.DMA((N_DEV - 1,)),
        ],
        compiler_params=pltpu.CompilerParams(collective_id=0),
    )(x, router_W, route_idx, expert_W)


# device time: 12492 ns/iter; 1.2370x vs baseline; 1.2370x over previous
import jax
import jax.numpy as jnp
from jax import lax
from jax.experimental import pallas as pl
from jax.experimental.pallas import tpu as pltpu

N_DEV = 4
E_PER = 2
N_EXP = N_DEV * E_PER


def kernel(x, router_W, route_idx, expert_W):
    n_tok, d = x.shape
    e_per, _, h = expert_W.shape

    def body(x_ref, rw_ref, idx_ref, ew_ref, out_ref,
             own_ref, comm_ref, send_sems, recv_sems, exit_sem):
        my = lax.axis_index("i")
        peers = [lax.rem(my + delta, N_DEV) for delta in (1, 2, 3)]

        barrier_sem = pltpu.get_barrier_semaphore()
        for peer in peers:
            pl.semaphore_signal(
                barrier_sem, inc=1,
                device_id=(peer,), device_id_type=pl.DeviceIdType.MESH,
            )
        pl.semaphore_wait(barrier_sem, N_DEV - 1)

        own_ref[...] = ew_ref[...].astype(jnp.bfloat16)

        rdma_by_slot = {}
        for delta in (1, 3, 2):
            slot = 3 - delta
            rdma = pltpu.make_async_remote_copy(
                src_ref=own_ref,
                dst_ref=comm_ref.at[slot],
                send_sem=send_sems.at[slot],
                recv_sem=recv_sems.at[slot],
                device_id=(lax.rem(my + delta, N_DEV),),
                device_id_type=pl.DeviceIdType.MESH,
            )
            rdma.start()
            rdma_by_slot[slot] = rdma

        scores = jnp.dot(x_ref[...], rw_ref[...],
                         preferred_element_type=jnp.float32)
        probs = jax.nn.softmax(scores, axis=-1)
        eids = lax.broadcasted_iota(jnp.int32, (n_tok, N_EXP), 1)
        mask = (eids == idx_ref[:, 0:1]) | (eids == idx_ref[:, 1:2])
        mp = jnp.where(mask, probs, 0.0)
        w = mp / jnp.sum(mp, axis=-1, keepdims=True)

        xb = x_ref[...].astype(jnp.bfloat16)

        def contrib(origin, shard):
            g0 = jnp.sum(jnp.where(eids == 2 * origin, w, 0.0),
                         axis=-1, keepdims=True)
            g1 = jnp.sum(jnp.where(eids == 2 * origin + 1, w, 0.0),
                         axis=-1, keepdims=True)
            y0 = jnp.dot(xb, shard[0], preferred_element_type=jnp.float32)
            y1 = jnp.dot(xb, shard[1], preferred_element_type=jnp.float32)
            return g0 * y0 + g1 * y1

        acc = contrib(my, own_ref[...])
        for slot in (2, 0, 1):
            rdma_by_slot[slot].wait_recv()
            acc = acc + contrib(lax.rem(my + slot + 1, N_DEV),
                                comm_ref[slot])
        out_ref[...] = acc

        for slot in (2, 0, 1):
            rdma_by_slot[slot].wait_send()

        for peer in peers:
            pl.semaphore_signal(
                exit_sem, inc=1,
                device_id=(peer,), device_id_type=pl.DeviceIdType.MESH,
            )
        pl.semaphore_wait(exit_sem, N_DEV - 1)

    return pl.pallas_call(
        body,
        out_shape=jax.ShapeDtypeStruct((n_tok, h), jnp.float32),
        in_specs=[pl.BlockSpec(memory_space=pltpu.VMEM)] * 4,
        out_specs=pl.BlockSpec(memory_space=pltpu.VMEM),
        scratch_shapes=[
            pltpu.VMEM((e_per, d, h), jnp.bfloat16),
            pltpu.VMEM((N_DEV - 1, e_per, d, h), jnp.bfloat16),
            pltpu.SemaphoreType.DMA((N_DEV - 1,)),
            pltpu.SemaphoreType.DMA((N_DEV - 1,)),
            pltpu.SemaphoreType.REGULAR,
        ],
        compiler_params=pltpu.CompilerParams(collective_id=0),
    )(x, router_W, route_idx, expert_W)


# device time: 10706 ns/iter; 1.4433x vs baseline; 1.1668x over previous
import jax
import jax.numpy as jnp
from jax import lax
from jax.experimental import pallas as pl
from jax.experimental.pallas import tpu as pltpu

N_DEV = 4
E_PER = 2
N_EXP = N_DEV * E_PER


def kernel(x, router_W, route_idx, expert_W):
    n_tok, d = x.shape
    e_per, _, h = expert_W.shape

    def body(x_ref, rw_ref, idx_ref, ew_ref, out_ref,
             own_ref, comm_ref, send_sems, recv_sems):
        my = lax.axis_index("i")
        peers = [lax.rem(my + delta, N_DEV) for delta in (1, 2, 3)]

        own_ref[...] = ew_ref[...].astype(jnp.bfloat16)

        barrier_sem = pltpu.get_barrier_semaphore()
        for peer in peers:
            pl.semaphore_signal(
                barrier_sem, inc=1,
                device_id=(peer,), device_id_type=pl.DeviceIdType.MESH,
            )
        pl.semaphore_wait(barrier_sem, N_DEV - 1)

        rdma_by_slot = {}
        for delta in (1, 3, 2):
            slot = 3 - delta
            rdma = pltpu.make_async_remote_copy(
                src_ref=own_ref,
                dst_ref=comm_ref.at[slot],
                send_sem=send_sems.at[slot],
                recv_sem=recv_sems.at[slot],
                device_id=(lax.rem(my + delta, N_DEV),),
                device_id_type=pl.DeviceIdType.MESH,
            )
            rdma.start()
            rdma_by_slot[slot] = rdma

        scores = jnp.dot(x_ref[...], rw_ref[...],
                         preferred_element_type=jnp.float32)
        probs = jax.nn.softmax(scores, axis=-1)
        eids = lax.broadcasted_iota(jnp.int32, (n_tok, N_EXP), 1)
        mask = (eids == idx_ref[:, 0:1]) | (eids == idx_ref[:, 1:2])
        mp = jnp.where(mask, probs, 0.0)
        w = mp / jnp.sum(mp, axis=-1, keepdims=True)

        xb = x_ref[...].astype(jnp.bfloat16)

        def contrib(origin, shard):
            g0 = jnp.sum(jnp.where(eids == 2 * origin, w, 0.0),
                         axis=-1, keepdims=True)
            g1 = jnp.sum(jnp.where(eids == 2 * origin + 1, w, 0.0),
                         axis=-1, keepdims=True)
            y0 = jnp.dot(xb, shard[0], preferred_element_type=jnp.float32)
            y1 = jnp.dot(xb, shard[1], preferred_element_type=jnp.float32)
            return g0 * y0 + g1 * y1

        acc = contrib(my, own_ref[...])
        for slot in (2, 0, 1):
            rdma_by_slot[slot].wait_recv()
            acc = acc + contrib(lax.rem(my + slot + 1, N_DEV),
                                comm_ref[slot])
        out_ref[...] = acc

        for slot in (2, 0, 1):
            rdma_by_slot[slot].wait_send()

    return pl.pallas_call(
        body,
        out_shape=jax.ShapeDtypeStruct((n_tok, h), jnp.float32),
        in_specs=[pl.BlockSpec(memory_space=pltpu.VMEM)] * 4,
        out_specs=pl.BlockSpec(memory_space=pltpu.VMEM),
        scratch_shapes=[
            pltpu.VMEM((e_per, d, h), jnp.bfloat16),
            pltpu.VMEM((N_DEV - 1, e_per, d, h), jnp.bfloat16),
            pltpu.SemaphoreType.DMA((N_DEV - 1,)),
            pltpu.SemaphoreType.DMA((N_DEV - 1,)),
        ],
        compiler_params=pltpu.CompilerParams(collective_id=0),
    )(x, router_W, route_idx, expert_W)
